# Initial kernel scaffold; baseline (speedup 1.0000x reference)
import jax
import jax.numpy as jnp
from jax.experimental import pallas as pl


def kernel(keys, values, indices, importance, access_times, access_frequency, importance_scores, global_time):
    raise NotImplementedError("write your pallas kernel here")


# speedup vs baseline: 7.0448x; 7.0448x over previous
import functools

import jax
import jax.numpy as jnp
from jax import lax
from jax.experimental import pallas as pl
from jax.experimental.pallas import tpu as pltpu
from jax.experimental.pallas import tpu_sc as plsc

_FREQ_W = 0.3
_IMP_W = 0.4
_TIME_W = 0.3
_NW = 32
_L = 16


def _mean_body(x_ref, o_ref):
    x = x_ref[0]
    o_ref[0, 0] = jnp.sum(x, axis=-1) * jnp.float32(1.0 / 128.0)


def _row_means(importance):
    b, h = importance.shape
    g = 16
    rows = b // g
    x3 = importance.reshape(g, rows, h)
    out = pl.pallas_call(
        _mean_body,
        grid=(g,),
        in_specs=[pl.BlockSpec((1, rows, h), lambda i: (i, 0, 0))],
        out_specs=pl.BlockSpec((1, 1, rows), lambda i: (i, 0, 0)),
        out_shape=jax.ShapeDtypeStruct((g, 1, rows), jnp.float32),
    )(x3)
    return out.reshape(b)


def _make_scatter_kernel(cp, b):
    r = cp // _NW
    groups = b // _L
    mesh = plsc.VectorSubcoreMesh(core_axis_name="c", subcore_axis_name="s")

    @functools.partial(
        pl.kernel,
        mesh=mesh,
        compiler_params=pltpu.CompilerParams(needs_layout_passes=False),
        out_type=[jax.ShapeDtypeStruct((cp,), jnp.float32)] * 3,
        scratch_types=[
            pltpu.VMEM((b,), jnp.int32),
            pltpu.VMEM((b,), jnp.float32),
            pltpu.VMEM((_L,), jnp.float32),
            pltpu.VMEM((r,), jnp.float32),
            pltpu.VMEM((r,), jnp.float32),
            pltpu.VMEM((r,), jnp.float32),
            pltpu.VMEM((r,), jnp.float32),
        ],
    )
    def scatter_kernel(idx_hbm, mean_hbm, gt_hbm, t_hbm, f_hbm, i_hbm,
                       t_out, f_out, i_out,
                       idx_v, mean_v, gt_v, t_v, f_v, f_old_v, i_v):
        wid = lax.axis_index("s") * 2 + lax.axis_index("c")
        lo = wid * r
        pltpu.sync_copy(idx_hbm, idx_v)
        pltpu.sync_copy(mean_hbm, mean_v)
        pltpu.sync_copy(gt_hbm, gt_v)
        pltpu.sync_copy(t_hbm.at[pl.ds(lo, r)], t_v)
        pltpu.sync_copy(f_hbm.at[pl.ds(lo, r)], f_v)
        pltpu.sync_copy(f_hbm.at[pl.ds(lo, r)], f_old_v)
        pltpu.sync_copy(i_hbm.at[pl.ds(lo, r)], i_v)
        gt16 = gt_v[...]

        def body(g, carry):
            ii = idx_v[pl.ds(g * _L, _L)]
            m = (ii >= lo) & (ii < lo + r)
            locs = jnp.where(m, ii - lo, 0)
            plsc.store_scatter(t_v, [locs], gt16, mask=m)
            old = plsc.load_gather(f_old_v, [locs], mask=m)
            plsc.store_scatter(f_v, [locs], old + 1.0, mask=m)
            mu = mean_v[pl.ds(g * _L, _L)]
            plsc.store_scatter(i_v, [locs], mu, mask=m)
            return carry

        lax.fori_loop(0, groups, body, 0)
        pltpu.sync_copy(t_v, t_out.at[pl.ds(lo, r)])
        pltpu.sync_copy(f_v, f_out.at[pl.ds(lo, r)])
        pltpu.sync_copy(i_v, i_out.at[pl.ds(lo, r)])

    return scatter_kernel


def _make_select_body(c, k, rows):
    def body(denom_ref, t_ref, f_ref, i_ref, p_ref, m_ref):
        t = t_ref[...]
        f = f_ref[...]
        im = i_ref[...]
        row = lax.broadcasted_iota(jnp.int32, (rows, 128), 0)
        col = lax.broadcasted_iota(jnp.int32, (rows, 128), 1)
        flat = row * 128 + col
        valid = flat < c

        fmax = jnp.max(f)
        imax = jnp.max(im)
        ts = t / denom_ref[0, 0]
        fs = f / (fmax + jnp.float32(1e-8))
        isc = im / (imax + jnp.float32(1e-8))
        p = jnp.float32(_TIME_W) * ts + jnp.float32(_FREQ_W) * fs \
            + jnp.float32(_IMP_W) * isc
        p = jnp.where(valid, p, jnp.float32(jnp.inf))
        p_ref[...] = p
        pbits = lax.bitcast_convert_type(p, jnp.int32)

        def cnt_le(bound):
            return jnp.sum((pbits <= bound).astype(jnp.int32))

        maxfinite = jnp.int32(0x7F7FFFFF)

        def bs_body(_, state):
            lo_, hi_ = state
            mid = (lo_ + hi_) >> 1
            take = cnt_le(mid) >= k
            return jnp.where(take, lo_, mid), jnp.where(take, mid, hi_)

        _, thr = lax.fori_loop(0, 31, bs_body, (jnp.int32(-1), maxfinite))
        c1 = cnt_le(thr - 1)
        need = k - c1
        eq = (pbits == thr) & valid

        def cnt2(bound):
            return jnp.sum((eq & (flat <= bound)).astype(jnp.int32))

        def bs2_body(_, state):
            lo_, hi_ = state
            mid = (lo_ + hi_) >> 1
            take = cnt2(mid) >= need
            return jnp.where(take, lo_, mid), jnp.where(take, mid, hi_)

        _, jthr = lax.fori_loop(0, 18, bs2_body,
                                (jnp.int32(-1), jnp.int32(rows * 128)))
        sel = (pbits < thr) | (eq & (flat <= jthr))
        m_ref[...] = sel.astype(jnp.int32)

    return body


def kernel(keys, values, indices, importance, access_times, access_frequency,
           importance_scores, global_time):
    cache_len = keys.shape[0]
    c = access_times.shape[0]
    if cache_len <= c:
        return jnp.zeros((cache_len,), dtype=jnp.bool_)
    b = indices.shape[0]
    k = cache_len - c
    del keys, values

    means = importance if importance.ndim == 1 else _row_means(importance)

    cp = ((c + 1023) // 1024) * 1024
    pad = cp - c
    gt_f = jnp.asarray(global_time).astype(jnp.float32)
    gt_vec = jnp.broadcast_to(gt_f, (_L,))
    idx_i32 = indices.astype(jnp.int32)
    t_p = jnp.concatenate([access_times, jnp.zeros((pad,), jnp.float32)])
    f_p = jnp.concatenate([access_frequency, jnp.zeros((pad,), jnp.float32)])
    i_p = jnp.concatenate([importance_scores, jnp.zeros((pad,), jnp.float32)])

    scatter = _make_scatter_kernel(cp, b)
    t_new, f_new, i_new = scatter(idx_i32, means, gt_vec, t_p, f_p, i_p)

    rows = cp // 128
    denom_t = (jnp.asarray(global_time) + 1).astype(jnp.float32) \
        + jnp.float32(1e-8)
    denom_t = denom_t.reshape(1, 1)
    p_grid, m_grid = pl.pallas_call(
        _make_select_body(c, k, rows),
        in_specs=[
            pl.BlockSpec(memory_space=pltpu.SMEM),
            pl.BlockSpec((rows, 128), lambda: (0, 0)),
            pl.BlockSpec((rows, 128), lambda: (0, 0)),
            pl.BlockSpec((rows, 128), lambda: (0, 0)),
        ],
        out_specs=[
            pl.BlockSpec((rows, 128), lambda: (0, 0)),
            pl.BlockSpec((rows, 128), lambda: (0, 0)),
        ],
        out_shape=[
            jax.ShapeDtypeStruct((rows, 128), jnp.float32),
            jax.ShapeDtypeStruct((rows, 128), jnp.int32),
        ],
    )(denom_t, t_new.reshape(rows, 128), f_new.reshape(rows, 128),
      i_new.reshape(rows, 128))

    priority = p_grid.reshape(cp)[:c]
    evict_mask = jnp.concatenate(
        [m_grid.reshape(cp)[:c], jnp.zeros((cache_len - c,), jnp.int32)]
    ).astype(jnp.bool_)
    return (evict_mask, priority, t_new[:c], f_new[:c], i_new[:c])
